# Initial kernel scaffold; baseline (speedup 1.0000x reference)
import jax
import jax.numpy as jnp
from jax.experimental import pallas as pl


def kernel(ent_emb, rel_emb, edge_index, rel_id, neigh_w):
    raise NotImplementedError("write your pallas kernel here")


# speedup vs baseline: 3.1537x; 3.1537x over previous
import functools
import jax
import jax.numpy as jnp
from jax import lax
from jax.experimental import pallas as pl
from jax.experimental.pallas import tpu as pltpu
from jax.experimental.pallas import tpu_sc as plsc

N_ENT = 10000
H = 128
N_REL = 475
N_EDGE = 320000

NCORE = 2
NSUB = 16
NW = NCORE * NSUB
EPW = 10240
E_PAD = NW * EPW
SEG_PAD = 10240
C = 128
NCH = EPW // C
COLS = SEG_PAD // NW
RPS = SEG_PAD // NSUB

_mesh = plsc.VectorSubcoreMesh(core_axis_name="c", subcore_axis_name="s")


def _wid():
    return lax.axis_index("s") * NCORE + lax.axis_index("c")


def _perm(v, idx):
    return v.at[idx].get(mode="promise_in_bounds")


def _seg_accum(tab, dv, uv, is_max):
    k, v = plsc.sort_key_val(dv, uv)
    iota = lax.iota(jnp.int32, 16)
    for off in (1, 2, 4, 8):
        idx = jnp.maximum(iota - off, 0)
        kp = _perm(k, idx)
        vp = _perm(v, idx)
        same = (kp == k) & (iota >= off)
        cmb = jnp.maximum(v, vp) if is_max else v + vp
        v = jnp.where(same, cmb, v)
    knext = _perm(k, jnp.minimum(iota + 1, 15))
    last = (knext != k) | (iota == 15)
    cur = plsc.load_gather(tab, [k])
    new = jnp.maximum(cur, v) if is_max else cur + v
    plsc.store_scatter(tab, [k], new, mask=last)


@functools.partial(
    pl.kernel,
    out_type=[
        jax.ShapeDtypeStruct((E_PAD,), jnp.float32),
        jax.ShapeDtypeStruct((NW * SEG_PAD,), jnp.float32),
    ],
    mesh=_mesh,
    compiler_params=pltpu.CompilerParams(needs_layout_passes=False),
    scratch_types=[
        pltpu.VMEM((C,), jnp.int32),
        pltpu.VMEM((C,), jnp.int32),
        pltpu.VMEM((C,), jnp.int32),
        pltpu.VMEM((C, H), jnp.float32),
        pltpu.VMEM((C, H), jnp.float32),
        pltpu.VMEM((C, H), jnp.float32),
        pltpu.VMEM((C,), jnp.float32),
        pltpu.VMEM((SEG_PAD,), jnp.float32),
        pltpu.SemaphoreType.DMA,
    ],
)
def _k1(src_h, dst_h, rel_h, ent_h, rele_h, norm_h, pmax_h,
        srcb, dstb, relb, bs, br, bd, nb, pmaxu, sem):
    wid = _wid()

    def zero(i, _):
        pmaxu[pl.ds(i * 16, 16)] = jnp.full((16,), -3.0e38, jnp.float32)
        return _
    lax.fori_loop(0, SEG_PAD // 16, zero, None)

    def chunk(ci, _):
        base = wid * EPW + ci * C
        pltpu.sync_copy(src_h.at[pl.ds(base, C)], srcb)
        pltpu.sync_copy(dst_h.at[pl.ds(base, C)], dstb)
        pltpu.sync_copy(rel_h.at[pl.ds(base, C)], relb)
        c1 = pltpu.async_copy(ent_h.at[srcb], bs, sem)
        c2 = pltpu.async_copy(rele_h.at[relb], br, sem)
        c3 = pltpu.async_copy(ent_h.at[dstb], bd, sem)
        c1.wait()
        c2.wait()
        c3.wait()

        def grp(g, _):
            def edge(j, nvec):
                e = g * 16 + j
                acc = jnp.zeros((16,), jnp.float32)
                for k in range(H // 16):
                    s = bs[e, pl.ds(k * 16, 16)]
                    r = br[e, pl.ds(k * 16, 16)]
                    d = bd[e, pl.ds(k * 16, 16)]
                    acc = acc + (s + r) * d
                iota = lax.iota(jnp.int32, 16)
                for off in (8, 4, 2, 1):
                    acc = acc + _perm(acc, (iota + off) & 15)
                return jnp.where(iota == j, acc, nvec)
            nvec = lax.fori_loop(0, 16, edge, jnp.zeros((16,), jnp.float32))
            nb[pl.ds(g * 16, 16)] = nvec
            _seg_accum(pmaxu, dstb[pl.ds(g * 16, 16)], nvec, True)
            return _
        lax.fori_loop(0, C // 16, grp, None)

        pltpu.sync_copy(nb, norm_h.at[pl.ds(base, C)])
        return _
    lax.fori_loop(0, NCH, chunk, None)
    pltpu.sync_copy(pmaxu, pmax_h.at[pl.ds(wid * SEG_PAD, SEG_PAD)])


def _make_reduce(is_max):
    @functools.partial(
        pl.kernel,
        out_type=jax.ShapeDtypeStruct((SEG_PAD,), jnp.float32),
        mesh=_mesh,
        compiler_params=pltpu.CompilerParams(needs_layout_passes=False),
        scratch_types=[
            pltpu.VMEM((NW * COLS,), jnp.float32),
            pltpu.VMEM((COLS,), jnp.float32),
        ],
    )
    def _red(part_h, out_h, buf, ob):
        wid = _wid()
        c0 = wid * COLS

        def row(r, _):
            pltpu.sync_copy(part_h.at[pl.ds(r * SEG_PAD + c0, COLS)],
                            buf.at[pl.ds(r * COLS, COLS)])
            return _
        lax.fori_loop(0, NW, row, None)

        def col(g, _):
            acc = buf[pl.ds(g * 16, 16)]
            for r in range(1, NW):
                v = buf[pl.ds(r * COLS + g * 16, 16)]
                acc = jnp.maximum(acc, v) if is_max else acc + v
            ob[pl.ds(g * 16, 16)] = acc
            return _
        lax.fori_loop(0, COLS // 16, col, None)
        pltpu.sync_copy(ob, out_h.at[pl.ds(c0, COLS)])

    return _red


_k2 = _make_reduce(True)
_k4 = _make_reduce(False)


@functools.partial(
    pl.kernel,
    out_type=[
        jax.ShapeDtypeStruct((E_PAD,), jnp.float32),
        jax.ShapeDtypeStruct((NW * SEG_PAD,), jnp.float32),
    ],
    mesh=_mesh,
    compiler_params=pltpu.CompilerParams(needs_layout_passes=False),
    scratch_types=[
        pltpu.VMEM((C,), jnp.int32),
        pltpu.VMEM((C,), jnp.float32),
        pltpu.VMEM((C,), jnp.float32),
        pltpu.VMEM((SEG_PAD,), jnp.float32),
        pltpu.VMEM((SEG_PAD,), jnp.float32),
    ],
)
def _k3(dst_h, norm_h, max_h, ex_h, psum_h, dstb, nb, exb, mtab, psum):
    wid = _wid()
    pltpu.sync_copy(max_h, mtab)

    def zero(i, _):
        psum[pl.ds(i * 16, 16)] = jnp.zeros((16,), jnp.float32)
        return _
    lax.fori_loop(0, SEG_PAD // 16, zero, None)

    def chunk(ci, _):
        base = wid * EPW + ci * C
        pltpu.sync_copy(dst_h.at[pl.ds(base, C)], dstb)
        pltpu.sync_copy(norm_h.at[pl.ds(base, C)], nb)

        def grp(g, _):
            dv = dstb[pl.ds(g * 16, 16)]
            mv = plsc.load_gather(mtab, [dv])
            nv = nb[pl.ds(g * 16, 16)]
            ev = jnp.exp(nv - mv)
            exb[pl.ds(g * 16, 16)] = ev
            _seg_accum(psum, dv, ev, False)
            return _
        lax.fori_loop(0, C // 16, grp, None)

        pltpu.sync_copy(exb, ex_h.at[pl.ds(base, C)])
        return _
    lax.fori_loop(0, NCH, chunk, None)
    pltpu.sync_copy(psum, psum_h.at[pl.ds(wid * SEG_PAD, SEG_PAD)])


@functools.partial(
    pl.kernel,
    out_type=jax.ShapeDtypeStruct((NCORE, SEG_PAD, H), jnp.float32),
    mesh=_mesh,
    compiler_params=pltpu.CompilerParams(needs_layout_passes=False),
    scratch_types=[
        pltpu.VMEM((C,), jnp.int32),
        pltpu.VMEM((C,), jnp.int32),
        pltpu.VMEM((C,), jnp.int32),
        pltpu.VMEM((C,), jnp.float32),
        pltpu.VMEM((C, H), jnp.float32),
        pltpu.VMEM((C, H), jnp.float32),
        pltpu.VMEM((SEG_PAD,), jnp.float32),
        pltpu.VMEM_SHARED((SEG_PAD, H), jnp.float32),
        pltpu.SemaphoreType.DMA,
    ],
)
def _k5(src_h, dst_h, rel_h, ex_h, ssum_h, ent_h, rele_h, out_h,
        srcb, dstb, relb, exb, bs, br, stab, acc, sem):
    cid = lax.axis_index("c")
    sid = lax.axis_index("s")
    wid = sid * NCORE + cid

    def zrow(i, _):
        for k in range(H // 16):
            bs[i, pl.ds(k * 16, 16)] = jnp.zeros((16,), jnp.float32)
        return _
    lax.fori_loop(0, C, zrow, None)
    for j in range(RPS // C):
        pltpu.sync_copy(bs, acc.at[pl.ds(sid * RPS + j * C, C), :])
    plsc.subcore_barrier()

    pltpu.sync_copy(ssum_h, stab)

    def chunk(ci, _):
        base = wid * EPW + ci * C
        pltpu.sync_copy(src_h.at[pl.ds(base, C)], srcb)
        pltpu.sync_copy(dst_h.at[pl.ds(base, C)], dstb)
        pltpu.sync_copy(rel_h.at[pl.ds(base, C)], relb)
        pltpu.sync_copy(ex_h.at[pl.ds(base, C)], exb)
        c1 = pltpu.async_copy(ent_h.at[srcb], bs, sem)
        c2 = pltpu.async_copy(rele_h.at[relb], br, sem)
        c1.wait()
        c2.wait()

        def grp(g, _):
            dv = dstb[pl.ds(g * 16, 16)]
            sv = plsc.load_gather(stab, [dv])
            alv = exb[pl.ds(g * 16, 16)] / sv

            def edge(j, _):
                e = g * 16 + j
                a = _perm(alv, jnp.zeros((16,), jnp.int32) + j)
                for k in range(H // 16):
                    s = bs[e, pl.ds(k * 16, 16)]
                    r = br[e, pl.ds(k * 16, 16)]
                    bs[e, pl.ds(k * 16, 16)] = (s + r) * a
                return _
            lax.fori_loop(0, 16, edge, None)
            return _
        lax.fori_loop(0, C // 16, grp, None)

        pltpu.sync_copy(bs, acc.at[dstb], add=True)
        return _
    lax.fori_loop(0, NCH, chunk, None)

    plsc.subcore_barrier()
    pltpu.sync_copy(acc.at[pl.ds(sid * RPS, RPS), :],
                    out_h.at[cid, pl.ds(sid * RPS, RPS), :])


def _mm_body(a_ref, b_ref, w_ref, o_ref):
    x = a_ref[...] + b_ref[...]
    o_ref[...] = jnp.tanh(jnp.dot(x, w_ref[...],
                                  preferred_element_type=jnp.float32))


def _tc_mm(p0, p1, w):
    blk = 1024
    return pl.pallas_call(
        _mm_body,
        grid=(SEG_PAD // blk,),
        in_specs=[
            pl.BlockSpec((blk, H), lambda i: (i, 0)),
            pl.BlockSpec((blk, H), lambda i: (i, 0)),
            pl.BlockSpec((H, H), lambda i: (0, 0)),
        ],
        out_specs=pl.BlockSpec((blk, H), lambda i: (i, 0)),
        out_shape=jax.ShapeDtypeStruct((SEG_PAD, H), jnp.float32),
    )(p0, p1, w)


@jax.jit
def kernel(ent_emb, rel_emb, edge_index, rel_id, neigh_w):
    src = edge_index[0].astype(jnp.int32)
    dst = edge_index[1].astype(jnp.int32)
    rel = rel_id.astype(jnp.int32)
    npad = E_PAD - N_EDGE
    src_p = jnp.concatenate([src, jnp.zeros((npad,), jnp.int32)])
    dst_p = jnp.concatenate([dst, jnp.full((npad,), N_ENT, jnp.int32)])
    rel_p = jnp.concatenate([rel, jnp.zeros((npad,), jnp.int32)])
    ent_p = jnp.pad(ent_emb, ((0, SEG_PAD - N_ENT), (0, 0)))

    norm, pmaxu = _k1(src_p, dst_p, rel_p, ent_p, rel_emb)
    segmax = _k2(pmaxu)
    ex, psum = _k3(dst_p, norm, segmax)
    segsum = _k4(psum)
    parts = _k5(src_p, dst_p, rel_p, ex, segsum, ent_p, rel_emb)
    out = _tc_mm(parts[0], parts[1], neigh_w)
    return out[:N_ENT]


# speedup vs baseline: 3.3295x; 1.0557x over previous
import functools
import jax
import jax.numpy as jnp
from jax import lax
from jax.experimental import pallas as pl
from jax.experimental.pallas import tpu as pltpu
from jax.experimental.pallas import tpu_sc as plsc

N_ENT = 10000
H = 128
N_REL = 475
N_EDGE = 320000

NCORE = 2
NSUB = 16
NW = NCORE * NSUB
EPW = 10240
E_PAD = NW * EPW
SEG_PAD = 10240
C = 128
NCH = EPW // C
COLS = SEG_PAD // NW
RPS = SEG_PAD // NSUB

_mesh = plsc.VectorSubcoreMesh(core_axis_name="c", subcore_axis_name="s")


def _wid():
    return lax.axis_index("s") * NCORE + lax.axis_index("c")


def _perm(v, idx):
    return v.at[idx].get(mode="promise_in_bounds")


def _seg_accum(tab, dv, uv, is_max):
    k, v = plsc.sort_key_val(dv, uv)
    iota = lax.iota(jnp.int32, 16)
    for off in (1, 2, 4, 8):
        idx = jnp.maximum(iota - off, 0)
        kp = _perm(k, idx)
        vp = _perm(v, idx)
        same = (kp == k) & (iota >= off)
        cmb = jnp.maximum(v, vp) if is_max else v + vp
        v = jnp.where(same, cmb, v)
    knext = _perm(k, jnp.minimum(iota + 1, 15))
    last = (knext != k) | (iota == 15)
    cur = plsc.load_gather(tab, [k])
    new = jnp.maximum(cur, v) if is_max else cur + v
    plsc.store_scatter(tab, [k], new, mask=last)


@functools.partial(
    pl.kernel,
    out_type=[
        jax.ShapeDtypeStruct((E_PAD,), jnp.float32),
        jax.ShapeDtypeStruct((NW * SEG_PAD,), jnp.float32),
    ],
    mesh=_mesh,
    compiler_params=pltpu.CompilerParams(needs_layout_passes=False),
    scratch_types=[
        pltpu.VMEM((C,), jnp.int32),
        pltpu.VMEM((C,), jnp.int32),
        pltpu.VMEM((C,), jnp.int32),
        pltpu.VMEM((C, H), jnp.float32),
        pltpu.VMEM((C, H), jnp.float32),
        pltpu.VMEM((C, H), jnp.float32),
        pltpu.VMEM((C,), jnp.float32),
        pltpu.VMEM((SEG_PAD,), jnp.float32),
        pltpu.SemaphoreType.DMA,
    ],
)
def _k1(src_h, dst_h, rel_h, ent_h, rele_h, norm_h, pmax_h,
        srcb, dstb, relb, bs, br, bd, nb, pmaxu, sem):
    wid = _wid()

    def zero(i, _):
        pmaxu[pl.ds(i * 16, 16)] = jnp.full((16,), -3.0e38, jnp.float32)
        return _
    lax.fori_loop(0, SEG_PAD // 16, zero, None)

    def chunk(ci, _):
        base = wid * EPW + ci * C
        pltpu.sync_copy(src_h.at[pl.ds(base, C)], srcb)
        pltpu.sync_copy(dst_h.at[pl.ds(base, C)], dstb)
        pltpu.sync_copy(rel_h.at[pl.ds(base, C)], relb)
        c1 = pltpu.async_copy(ent_h.at[srcb], bs, sem)
        c2 = pltpu.async_copy(rele_h.at[relb], br, sem)
        c3 = pltpu.async_copy(ent_h.at[dstb], bd, sem)
        c1.wait()
        c2.wait()
        c3.wait()

        def grp(g, _):
            iota = lax.iota(jnp.int32, 16)
            nvec = jnp.zeros((16,), jnp.float32)
            for j in range(16):
                e = g * 16 + j
                acc = jnp.zeros((16,), jnp.float32)
                for k in range(H // 16):
                    s = bs[e, pl.ds(k * 16, 16)]
                    r = br[e, pl.ds(k * 16, 16)]
                    d = bd[e, pl.ds(k * 16, 16)]
                    acc = acc + (s + r) * d
                for off in (8, 4, 2, 1):
                    acc = acc + _perm(acc, (iota + off) & 15)
                nvec = jnp.where(iota == j, acc, nvec)
            nb[pl.ds(g * 16, 16)] = nvec
            _seg_accum(pmaxu, dstb[pl.ds(g * 16, 16)], nvec, True)
            return _
        lax.fori_loop(0, C // 16, grp, None)

        pltpu.sync_copy(nb, norm_h.at[pl.ds(base, C)])
        return _
    lax.fori_loop(0, NCH, chunk, None)
    pltpu.sync_copy(pmaxu, pmax_h.at[pl.ds(wid * SEG_PAD, SEG_PAD)])


def _make_reduce(is_max):
    @functools.partial(
        pl.kernel,
        out_type=jax.ShapeDtypeStruct((SEG_PAD,), jnp.float32),
        mesh=_mesh,
        compiler_params=pltpu.CompilerParams(needs_layout_passes=False),
        scratch_types=[
            pltpu.VMEM((NW * COLS,), jnp.float32),
            pltpu.VMEM((COLS,), jnp.float32),
        ],
    )
    def _red(part_h, out_h, buf, ob):
        wid = _wid()
        c0 = wid * COLS

        def row(r, _):
            pltpu.sync_copy(part_h.at[pl.ds(r * SEG_PAD + c0, COLS)],
                            buf.at[pl.ds(r * COLS, COLS)])
            return _
        lax.fori_loop(0, NW, row, None)

        def col(g, _):
            acc = buf[pl.ds(g * 16, 16)]
            for r in range(1, NW):
                v = buf[pl.ds(r * COLS + g * 16, 16)]
                acc = jnp.maximum(acc, v) if is_max else acc + v
            ob[pl.ds(g * 16, 16)] = acc
            return _
        lax.fori_loop(0, COLS // 16, col, None)
        pltpu.sync_copy(ob, out_h.at[pl.ds(c0, COLS)])

    return _red


_k2 = _make_reduce(True)
_k4 = _make_reduce(False)


@functools.partial(
    pl.kernel,
    out_type=[
        jax.ShapeDtypeStruct((E_PAD,), jnp.float32),
        jax.ShapeDtypeStruct((NW * SEG_PAD,), jnp.float32),
    ],
    mesh=_mesh,
    compiler_params=pltpu.CompilerParams(needs_layout_passes=False),
    scratch_types=[
        pltpu.VMEM((C,), jnp.int32),
        pltpu.VMEM((C,), jnp.float32),
        pltpu.VMEM((C,), jnp.float32),
        pltpu.VMEM((SEG_PAD,), jnp.float32),
        pltpu.VMEM((SEG_PAD,), jnp.float32),
    ],
)
def _k3(dst_h, norm_h, max_h, ex_h, psum_h, dstb, nb, exb, mtab, psum):
    wid = _wid()
    pltpu.sync_copy(max_h, mtab)

    def zero(i, _):
        psum[pl.ds(i * 16, 16)] = jnp.zeros((16,), jnp.float32)
        return _
    lax.fori_loop(0, SEG_PAD // 16, zero, None)

    def chunk(ci, _):
        base = wid * EPW + ci * C
        pltpu.sync_copy(dst_h.at[pl.ds(base, C)], dstb)
        pltpu.sync_copy(norm_h.at[pl.ds(base, C)], nb)

        def grp(g, _):
            dv = dstb[pl.ds(g * 16, 16)]
            mv = plsc.load_gather(mtab, [dv])
            nv = nb[pl.ds(g * 16, 16)]
            ev = jnp.exp(nv - mv)
            exb[pl.ds(g * 16, 16)] = ev
            _seg_accum(psum, dv, ev, False)
            return _
        lax.fori_loop(0, C // 16, grp, None)

        pltpu.sync_copy(exb, ex_h.at[pl.ds(base, C)])
        return _
    lax.fori_loop(0, NCH, chunk, None)
    pltpu.sync_copy(psum, psum_h.at[pl.ds(wid * SEG_PAD, SEG_PAD)])


@functools.partial(
    pl.kernel,
    out_type=jax.ShapeDtypeStruct((NCORE, SEG_PAD, H), jnp.float32),
    mesh=_mesh,
    compiler_params=pltpu.CompilerParams(needs_layout_passes=False),
    scratch_types=[
        pltpu.VMEM((C,), jnp.int32),
        pltpu.VMEM((C,), jnp.int32),
        pltpu.VMEM((C,), jnp.int32),
        pltpu.VMEM((C,), jnp.float32),
        pltpu.VMEM((C, H), jnp.float32),
        pltpu.VMEM((C, H), jnp.float32),
        pltpu.VMEM((SEG_PAD,), jnp.float32),
        pltpu.VMEM_SHARED((SEG_PAD, H), jnp.float32),
        pltpu.SemaphoreType.DMA,
    ],
)
def _k5(src_h, dst_h, rel_h, ex_h, ssum_h, ent_h, rele_h, out_h,
        srcb, dstb, relb, exb, bs, br, stab, acc, sem):
    cid = lax.axis_index("c")
    sid = lax.axis_index("s")
    wid = sid * NCORE + cid

    def zrow(i, _):
        for k in range(H // 16):
            bs[i, pl.ds(k * 16, 16)] = jnp.zeros((16,), jnp.float32)
        return _
    lax.fori_loop(0, C, zrow, None)
    for j in range(RPS // C):
        pltpu.sync_copy(bs, acc.at[pl.ds(sid * RPS + j * C, C), :])
    plsc.subcore_barrier()

    pltpu.sync_copy(ssum_h, stab)

    def chunk(ci, _):
        base = wid * EPW + ci * C
        pltpu.sync_copy(src_h.at[pl.ds(base, C)], srcb)
        pltpu.sync_copy(dst_h.at[pl.ds(base, C)], dstb)
        pltpu.sync_copy(rel_h.at[pl.ds(base, C)], relb)
        pltpu.sync_copy(ex_h.at[pl.ds(base, C)], exb)
        c1 = pltpu.async_copy(ent_h.at[srcb], bs, sem)
        c2 = pltpu.async_copy(rele_h.at[relb], br, sem)
        c1.wait()
        c2.wait()

        def grp(g, _):
            dv = dstb[pl.ds(g * 16, 16)]
            sv = plsc.load_gather(stab, [dv])
            alv = exb[pl.ds(g * 16, 16)] / sv

            for j in range(16):
                e = g * 16 + j
                a = _perm(alv, jnp.full((16,), j, jnp.int32))
                for k in range(H // 16):
                    s = bs[e, pl.ds(k * 16, 16)]
                    r = br[e, pl.ds(k * 16, 16)]
                    bs[e, pl.ds(k * 16, 16)] = (s + r) * a
            return _
        lax.fori_loop(0, C // 16, grp, None)

        pltpu.sync_copy(bs, acc.at[dstb], add=True)
        return _
    lax.fori_loop(0, NCH, chunk, None)

    plsc.subcore_barrier()
    pltpu.sync_copy(acc.at[pl.ds(sid * RPS, RPS), :],
                    out_h.at[cid, pl.ds(sid * RPS, RPS), :])


def _mm_body(a_ref, b_ref, w_ref, o_ref):
    x = a_ref[...] + b_ref[...]
    o_ref[...] = jnp.tanh(jnp.dot(x, w_ref[...],
                                  preferred_element_type=jnp.float32))


def _tc_mm(p0, p1, w):
    blk = 1024
    return pl.pallas_call(
        _mm_body,
        grid=(SEG_PAD // blk,),
        in_specs=[
            pl.BlockSpec((blk, H), lambda i: (i, 0)),
            pl.BlockSpec((blk, H), lambda i: (i, 0)),
            pl.BlockSpec((H, H), lambda i: (0, 0)),
        ],
        out_specs=pl.BlockSpec((blk, H), lambda i: (i, 0)),
        out_shape=jax.ShapeDtypeStruct((SEG_PAD, H), jnp.float32),
    )(p0, p1, w)


@jax.jit
def kernel(ent_emb, rel_emb, edge_index, rel_id, neigh_w):
    src = edge_index[0].astype(jnp.int32)
    dst = edge_index[1].astype(jnp.int32)
    rel = rel_id.astype(jnp.int32)
    npad = E_PAD - N_EDGE
    src_p = jnp.concatenate([src, jnp.zeros((npad,), jnp.int32)])
    dst_p = jnp.concatenate([dst, jnp.full((npad,), N_ENT, jnp.int32)])
    rel_p = jnp.concatenate([rel, jnp.zeros((npad,), jnp.int32)])
    ent_p = jnp.pad(ent_emb, ((0, SEG_PAD - N_ENT), (0, 0)))

    norm, pmaxu = _k1(src_p, dst_p, rel_p, ent_p, rel_emb)
    segmax = _k2(pmaxu)
    ex, psum = _k3(dst_p, norm, segmax)
    segsum = _k4(psum)
    parts = _k5(src_p, dst_p, rel_p, ex, segsum, ent_p, rel_emb)
    out = _tc_mm(parts[0], parts[1], neigh_w)
    return out[:N_ENT]


# speedup vs baseline: 3.5245x; 1.0586x over previous
import functools
import jax
import jax.numpy as jnp
from jax import lax
from jax.experimental import pallas as pl
from jax.experimental.pallas import tpu as pltpu
from jax.experimental.pallas import tpu_sc as plsc

N_ENT = 10000
H = 128
N_REL = 475
N_EDGE = 320000

NCORE = 2
NSUB = 16
NW = NCORE * NSUB
EPW = 10240
E_PAD = NW * EPW
SEG_PAD = 10240
C = 128
NCH = EPW // C
COLS = SEG_PAD // NW
RPS = SEG_PAD // NSUB

_mesh = plsc.VectorSubcoreMesh(core_axis_name="c", subcore_axis_name="s")


def _wid():
    return lax.axis_index("s") * NCORE + lax.axis_index("c")


def _perm(v, idx):
    return v.at[idx].get(mode="promise_in_bounds")


def _seg_accum(tab, dv, uv, is_max):
    k, v = plsc.sort_key_val(dv, uv)
    iota = lax.iota(jnp.int32, 16)
    for off in (1, 2, 4, 8):
        idx = jnp.maximum(iota - off, 0)
        kp = _perm(k, idx)
        vp = _perm(v, idx)
        same = (kp == k) & (iota >= off)
        cmb = jnp.maximum(v, vp) if is_max else v + vp
        v = jnp.where(same, cmb, v)
    knext = _perm(k, jnp.minimum(iota + 1, 15))
    last = (knext != k) | (iota == 15)
    cur = plsc.load_gather(tab, [k])
    new = jnp.maximum(cur, v) if is_max else cur + v
    plsc.store_scatter(tab, [k], new, mask=last)


S1 = 2048
CPS1 = S1 // C
NSC1 = EPW // S1


@functools.partial(
    pl.kernel,
    out_type=[
        jax.ShapeDtypeStruct((E_PAD,), jnp.float32),
        jax.ShapeDtypeStruct((NW * SEG_PAD,), jnp.float32),
    ],
    mesh=_mesh,
    compiler_params=pltpu.CompilerParams(needs_layout_passes=False),
    scratch_types=[
        pltpu.VMEM((S1,), jnp.int32),
        pltpu.VMEM((S1,), jnp.int32),
        pltpu.VMEM((S1,), jnp.int32),
        pltpu.VMEM((C, H), jnp.float32),
        pltpu.VMEM((C, H), jnp.float32),
        pltpu.VMEM((C, H), jnp.float32),
        pltpu.VMEM((C, H), jnp.float32),
        pltpu.VMEM((C, H), jnp.float32),
        pltpu.VMEM((C, H), jnp.float32),
        pltpu.VMEM((S1,), jnp.float32),
        pltpu.VMEM((SEG_PAD,), jnp.float32),
        pltpu.SemaphoreType.DMA,
        pltpu.SemaphoreType.DMA,
    ],
)
def _k1(src_h, dst_h, rel_h, ent_h, rele_h, norm_h, pmax_h,
        srcb, dstb, relb, bs0, br0, bd0, bs1, br1, bd1, nb, pmaxu,
        semA, semB):
    wid = _wid()

    def zero(i, _):
        pmaxu[pl.ds(i * 16, 16)] = jnp.full((16,), -3.0e38, jnp.float32)
        return _
    lax.fori_loop(0, SEG_PAD // 16, zero, None)

    def dot_chunk(bs, br, bd, off):
        def grp(g, _):
            iota = lax.iota(jnp.int32, 16)
            nvec = jnp.zeros((16,), jnp.float32)
            for j in range(16):
                e = g * 16 + j
                acc = jnp.zeros((16,), jnp.float32)
                for k in range(H // 16):
                    s = bs[e, pl.ds(k * 16, 16)]
                    r = br[e, pl.ds(k * 16, 16)]
                    d = bd[e, pl.ds(k * 16, 16)]
                    acc = acc + (s + r) * d
                for o in (8, 4, 2, 1):
                    acc = acc + _perm(acc, (iota + o) & 15)
                nvec = jnp.where(iota == j, acc, nvec)
            nb[pl.ds(off + g * 16, 16)] = nvec
            _seg_accum(pmaxu, dstb[pl.ds(off + g * 16, 16)], nvec, True)
            return _
        lax.fori_loop(0, C // 16, grp, None)

    def sch(si, _):
        sbase = wid * EPW + si * S1
        pltpu.sync_copy(src_h.at[pl.ds(sbase, S1)], srcb)
        pltpu.sync_copy(dst_h.at[pl.ds(sbase, S1)], dstb)
        pltpu.sync_copy(rel_h.at[pl.ds(sbase, S1)], relb)

        def pair(p, _):
            o0 = (2 * p) * C
            o1 = o0 + C
            a1 = pltpu.async_copy(ent_h.at[srcb.at[pl.ds(o0, C)]], bs0, semA)
            a2 = pltpu.async_copy(rele_h.at[relb.at[pl.ds(o0, C)]], br0, semA)
            a3 = pltpu.async_copy(ent_h.at[dstb.at[pl.ds(o0, C)]], bd0, semA)
            b1 = pltpu.async_copy(ent_h.at[srcb.at[pl.ds(o1, C)]], bs1, semB)
            b2 = pltpu.async_copy(rele_h.at[relb.at[pl.ds(o1, C)]], br1, semB)
            b3 = pltpu.async_copy(ent_h.at[dstb.at[pl.ds(o1, C)]], bd1, semB)
            a1.wait()
            a2.wait()
            a3.wait()
            dot_chunk(bs0, br0, bd0, o0)
            b1.wait()
            b2.wait()
            b3.wait()
            dot_chunk(bs1, br1, bd1, o1)
            return _
        lax.fori_loop(0, CPS1 // 2, pair, None)
        pltpu.sync_copy(nb, norm_h.at[pl.ds(sbase, S1)])
        return _
    lax.fori_loop(0, NSC1, sch, None)
    pltpu.sync_copy(pmaxu, pmax_h.at[pl.ds(wid * SEG_PAD, SEG_PAD)])


def _make_reduce(is_max):
    @functools.partial(
        pl.kernel,
        out_type=jax.ShapeDtypeStruct((SEG_PAD,), jnp.float32),
        mesh=_mesh,
        compiler_params=pltpu.CompilerParams(needs_layout_passes=False),
        scratch_types=[
            pltpu.VMEM((NW * COLS,), jnp.float32),
            pltpu.VMEM((COLS,), jnp.float32),
        ],
    )
    def _red(part_h, out_h, buf, ob):
        wid = _wid()
        c0 = wid * COLS

        def row(r, _):
            pltpu.sync_copy(part_h.at[pl.ds(r * SEG_PAD + c0, COLS)],
                            buf.at[pl.ds(r * COLS, COLS)])
            return _
        lax.fori_loop(0, NW, row, None)

        def col(g, _):
            acc = buf[pl.ds(g * 16, 16)]
            for r in range(1, NW):
                v = buf[pl.ds(r * COLS + g * 16, 16)]
                acc = jnp.maximum(acc, v) if is_max else acc + v
            ob[pl.ds(g * 16, 16)] = acc
            return _
        lax.fori_loop(0, COLS // 16, col, None)
        pltpu.sync_copy(ob, out_h.at[pl.ds(c0, COLS)])

    return _red


_k2 = _make_reduce(True)
_k4 = _make_reduce(False)


@functools.partial(
    pl.kernel,
    out_type=[
        jax.ShapeDtypeStruct((E_PAD,), jnp.float32),
        jax.ShapeDtypeStruct((NW * SEG_PAD,), jnp.float32),
    ],
    mesh=_mesh,
    compiler_params=pltpu.CompilerParams(needs_layout_passes=False),
    scratch_types=[
        pltpu.VMEM((C,), jnp.int32),
        pltpu.VMEM((C,), jnp.float32),
        pltpu.VMEM((C,), jnp.float32),
        pltpu.VMEM((SEG_PAD,), jnp.float32),
        pltpu.VMEM((SEG_PAD,), jnp.float32),
    ],
)
def _k3(dst_h, norm_h, max_h, ex_h, psum_h, dstb, nb, exb, mtab, psum):
    wid = _wid()
    pltpu.sync_copy(max_h, mtab)

    def zero(i, _):
        psum[pl.ds(i * 16, 16)] = jnp.zeros((16,), jnp.float32)
        return _
    lax.fori_loop(0, SEG_PAD // 16, zero, None)

    def chunk(ci, _):
        base = wid * EPW + ci * C
        pltpu.sync_copy(dst_h.at[pl.ds(base, C)], dstb)
        pltpu.sync_copy(norm_h.at[pl.ds(base, C)], nb)

        def grp(g, _):
            dv = dstb[pl.ds(g * 16, 16)]
            mv = plsc.load_gather(mtab, [dv])
            nv = nb[pl.ds(g * 16, 16)]
            ev = jnp.exp(nv - mv)
            exb[pl.ds(g * 16, 16)] = ev
            _seg_accum(psum, dv, ev, False)
            return _
        lax.fori_loop(0, C // 16, grp, None)

        pltpu.sync_copy(exb, ex_h.at[pl.ds(base, C)])
        return _
    lax.fori_loop(0, NCH, chunk, None)
    pltpu.sync_copy(psum, psum_h.at[pl.ds(wid * SEG_PAD, SEG_PAD)])


C5 = 64
S5 = 1024
CPS5 = S5 // C5
NSC5 = EPW // S5


@functools.partial(
    pl.kernel,
    out_type=jax.ShapeDtypeStruct((NCORE, SEG_PAD, H), jnp.float32),
    mesh=_mesh,
    compiler_params=pltpu.CompilerParams(needs_layout_passes=False),
    scratch_types=[
        pltpu.VMEM((S5,), jnp.int32),
        pltpu.VMEM((S5,), jnp.int32),
        pltpu.VMEM((S5,), jnp.float32),
        pltpu.VMEM((C5,), jnp.int32),
        pltpu.VMEM((C5,), jnp.int32),
        pltpu.VMEM((C5, H), jnp.float32),
        pltpu.VMEM((C5, H), jnp.float32),
        pltpu.VMEM((C5, H), jnp.float32),
        pltpu.VMEM((C5, H), jnp.float32),
        pltpu.VMEM((SEG_PAD,), jnp.float32),
        pltpu.VMEM_SHARED((SEG_PAD, H), jnp.float32),
        pltpu.SemaphoreType.DMA,
        pltpu.SemaphoreType.DMA,
        pltpu.SemaphoreType.DMA,
    ],
)
def _k5(src_h, dst_h, rel_h, ex_h, ssum_h, ent_h, rele_h, out_h,
        srcb, relb, exb, dstb0, dstb1, bs0, br0, bs1, br1, stab, acc,
        semA, semB, semS):
    cid = lax.axis_index("c")
    sid = lax.axis_index("s")
    wid = sid * NCORE + cid

    def zrow(i, _):
        for k in range(H // 16):
            bs0[i, pl.ds(k * 16, 16)] = jnp.zeros((16,), jnp.float32)
        return _
    lax.fori_loop(0, C5, zrow, None)
    for j in range(RPS // C5):
        pltpu.sync_copy(bs0, acc.at[pl.ds(sid * RPS + j * C5, C5), :])
    plsc.subcore_barrier()

    pltpu.sync_copy(ssum_h, stab)

    def comp_chunk(bs, br, dstb, off):
        def grp(g, _):
            dv = dstb[pl.ds(g * 16, 16)]
            sv = plsc.load_gather(stab, [dv])
            alv = exb[pl.ds(off + g * 16, 16)] / sv
            for j in range(16):
                e = g * 16 + j
                a = _perm(alv, jnp.full((16,), j, jnp.int32))
                for k in range(H // 16):
                    s = bs[e, pl.ds(k * 16, 16)]
                    r = br[e, pl.ds(k * 16, 16)]
                    bs[e, pl.ds(k * 16, 16)] = (s + r) * a
            return _
        lax.fori_loop(0, C5 // 16, grp, None)

    def sch(si, _):
        sbase = wid * EPW + si * S5
        pltpu.sync_copy(src_h.at[pl.ds(sbase, S5)], srcb)
        pltpu.sync_copy(rel_h.at[pl.ds(sbase, S5)], relb)
        pltpu.sync_copy(ex_h.at[pl.ds(sbase, S5)], exb)

        def pair(p, _):
            o0 = (2 * p) * C5
            o1 = o0 + C5
            a1 = pltpu.async_copy(ent_h.at[srcb.at[pl.ds(o0, C5)]], bs0, semA)
            a2 = pltpu.async_copy(rele_h.at[relb.at[pl.ds(o0, C5)]], br0, semA)
            a3 = pltpu.async_copy(dst_h.at[pl.ds(sbase + o0, C5)], dstb0, semA)
            b1 = pltpu.async_copy(ent_h.at[srcb.at[pl.ds(o1, C5)]], bs1, semB)
            b2 = pltpu.async_copy(rele_h.at[relb.at[pl.ds(o1, C5)]], br1, semB)
            b3 = pltpu.async_copy(dst_h.at[pl.ds(sbase + o1, C5)], dstb1, semB)
            a1.wait()
            a2.wait()
            a3.wait()
            comp_chunk(bs0, br0, dstb0, o0)
            s0 = pltpu.async_copy(bs0, acc.at[dstb0], semS, add=True)
            b1.wait()
            b2.wait()
            b3.wait()
            comp_chunk(bs1, br1, dstb1, o1)
            s1 = pltpu.async_copy(bs1, acc.at[dstb1], semS, add=True)
            s0.wait()
            s1.wait()
            return _
        lax.fori_loop(0, CPS5 // 2, pair, None)
        return _
    lax.fori_loop(0, NSC5, sch, None)

    plsc.subcore_barrier()
    pltpu.sync_copy(acc.at[pl.ds(sid * RPS, RPS), :],
                    out_h.at[cid, pl.ds(sid * RPS, RPS), :])


def _mm_body(a_ref, b_ref, w_ref, o_ref):
    x = a_ref[...] + b_ref[...]
    o_ref[...] = jnp.tanh(jnp.dot(x, w_ref[...],
                                  preferred_element_type=jnp.float32))


def _tc_mm(p0, p1, w):
    blk = 1024
    return pl.pallas_call(
        _mm_body,
        grid=(SEG_PAD // blk,),
        in_specs=[
            pl.BlockSpec((blk, H), lambda i: (i, 0)),
            pl.BlockSpec((blk, H), lambda i: (i, 0)),
            pl.BlockSpec((H, H), lambda i: (0, 0)),
        ],
        out_specs=pl.BlockSpec((blk, H), lambda i: (i, 0)),
        out_shape=jax.ShapeDtypeStruct((SEG_PAD, H), jnp.float32),
    )(p0, p1, w)


@jax.jit
def kernel(ent_emb, rel_emb, edge_index, rel_id, neigh_w):
    src = edge_index[0].astype(jnp.int32)
    dst = edge_index[1].astype(jnp.int32)
    rel = rel_id.astype(jnp.int32)
    npad = E_PAD - N_EDGE
    src_p = jnp.concatenate([src, jnp.zeros((npad,), jnp.int32)])
    dst_p = jnp.concatenate([dst, jnp.full((npad,), N_ENT, jnp.int32)])
    rel_p = jnp.concatenate([rel, jnp.zeros((npad,), jnp.int32)])
    ent_p = jnp.pad(ent_emb, ((0, SEG_PAD - N_ENT), (0, 0)))

    norm, pmaxu = _k1(src_p, dst_p, rel_p, ent_p, rel_emb)
    segmax = _k2(pmaxu)
    ex, psum = _k3(dst_p, norm, segmax)
    segsum = _k4(psum)
    parts = _k5(src_p, dst_p, rel_p, ex, segsum, ent_p, rel_emb)
    out = _tc_mm(parts[0], parts[1], neigh_w)
    return out[:N_ENT]


# speedup vs baseline: 3.6311x; 1.0302x over previous
import functools
import jax
import jax.numpy as jnp
from jax import lax
from jax.experimental import pallas as pl
from jax.experimental.pallas import tpu as pltpu
from jax.experimental.pallas import tpu_sc as plsc

N_ENT = 10000
H = 128
N_REL = 475
N_EDGE = 320000

NCORE = 2
NSUB = 16
NW = NCORE * NSUB
EPW = 10240
E_PAD = NW * EPW
SEG_PAD = 10240
C = 128
NCH = EPW // C
COLS = SEG_PAD // NW
RPS = SEG_PAD // NSUB

_mesh = plsc.VectorSubcoreMesh(core_axis_name="c", subcore_axis_name="s")


def _wid():
    return lax.axis_index("s") * NCORE + lax.axis_index("c")


def _perm(v, idx):
    return v.at[idx].get(mode="promise_in_bounds")


def _seg_accum(tab, dv, uv, is_max):
    k, v = plsc.sort_key_val(dv, uv)
    iota = lax.iota(jnp.int32, 16)
    for off in (1, 2, 4, 8):
        idx = jnp.maximum(iota - off, 0)
        kp = _perm(k, idx)
        vp = _perm(v, idx)
        same = (kp == k) & (iota >= off)
        cmb = jnp.maximum(v, vp) if is_max else v + vp
        v = jnp.where(same, cmb, v)
    knext = _perm(k, jnp.minimum(iota + 1, 15))
    last = (knext != k) | (iota == 15)
    cur = plsc.load_gather(tab, [k])
    new = jnp.maximum(cur, v) if is_max else cur + v
    plsc.store_scatter(tab, [k], new, mask=last)


S1 = 2048
CPS1 = S1 // C
NSC1 = EPW // S1


@functools.partial(
    pl.kernel,
    out_type=[
        jax.ShapeDtypeStruct((E_PAD,), jnp.float32),
        jax.ShapeDtypeStruct((NW * SEG_PAD,), jnp.float32),
    ],
    mesh=_mesh,
    compiler_params=pltpu.CompilerParams(needs_layout_passes=False),
    scratch_types=[
        pltpu.VMEM((S1,), jnp.int32),
        pltpu.VMEM((S1,), jnp.int32),
        pltpu.VMEM((S1,), jnp.int32),
        pltpu.VMEM((C, H), jnp.float32),
        pltpu.VMEM((C, H), jnp.float32),
        pltpu.VMEM((C, H), jnp.float32),
        pltpu.VMEM((C, H), jnp.float32),
        pltpu.VMEM((C, H), jnp.float32),
        pltpu.VMEM((C, H), jnp.float32),
        pltpu.VMEM((S1,), jnp.float32),
        pltpu.VMEM((SEG_PAD,), jnp.float32),
        pltpu.SemaphoreType.DMA,
        pltpu.SemaphoreType.DMA,
    ],
)
def _k1(src_h, dst_h, rel_h, ent_h, rele_h, norm_h, pmax_h,
        srcb, dstb, relb, bs0, br0, bd0, bs1, br1, bd1, nb, pmaxu,
        semA, semB):
    wid = _wid()

    def zero(i, _):
        pmaxu[pl.ds(i * 16, 16)] = jnp.full((16,), -3.0e38, jnp.float32)
        return _
    lax.fori_loop(0, SEG_PAD // 16, zero, None)

    def dot_chunk(bs, br, bd, off):
        def grp(g, _):
            iota = lax.iota(jnp.int32, 16)
            nvec = jnp.zeros((16,), jnp.float32)
            for j in range(16):
                e = g * 16 + j
                acc = jnp.zeros((16,), jnp.float32)
                for k in range(H // 16):
                    s = bs[e, pl.ds(k * 16, 16)]
                    r = br[e, pl.ds(k * 16, 16)]
                    d = bd[e, pl.ds(k * 16, 16)]
                    acc = acc + (s + r) * d
                for o in (8, 4, 2, 1):
                    acc = acc + _perm(acc, (iota + o) & 15)
                nvec = jnp.where(iota == j, acc, nvec)
            nb[pl.ds(off + g * 16, 16)] = nvec
            _seg_accum(pmaxu, dstb[pl.ds(off + g * 16, 16)], nvec, True)
            return _
        lax.fori_loop(0, C // 16, grp, None)

    def sch(si, _):
        sbase = wid * EPW + si * S1
        pltpu.sync_copy(src_h.at[pl.ds(sbase, S1)], srcb)
        pltpu.sync_copy(dst_h.at[pl.ds(sbase, S1)], dstb)
        pltpu.sync_copy(rel_h.at[pl.ds(sbase, S1)], relb)

        def pair(p, _):
            o0 = (2 * p) * C
            o1 = o0 + C
            a1 = pltpu.async_copy(ent_h.at[srcb.at[pl.ds(o0, C)]], bs0, semA)
            a2 = pltpu.async_copy(rele_h.at[relb.at[pl.ds(o0, C)]], br0, semA)
            a3 = pltpu.async_copy(ent_h.at[dstb.at[pl.ds(o0, C)]], bd0, semA)
            b1 = pltpu.async_copy(ent_h.at[srcb.at[pl.ds(o1, C)]], bs1, semB)
            b2 = pltpu.async_copy(rele_h.at[relb.at[pl.ds(o1, C)]], br1, semB)
            b3 = pltpu.async_copy(ent_h.at[dstb.at[pl.ds(o1, C)]], bd1, semB)
            a1.wait()
            a2.wait()
            a3.wait()
            dot_chunk(bs0, br0, bd0, o0)
            b1.wait()
            b2.wait()
            b3.wait()
            dot_chunk(bs1, br1, bd1, o1)
            return _
        lax.fori_loop(0, CPS1 // 2, pair, None)
        pltpu.sync_copy(nb, norm_h.at[pl.ds(sbase, S1)])
        return _
    lax.fori_loop(0, NSC1, sch, None)
    pltpu.sync_copy(pmaxu, pmax_h.at[pl.ds(wid * SEG_PAD, SEG_PAD)])


def _make_reduce(is_max):
    @functools.partial(
        pl.kernel,
        out_type=jax.ShapeDtypeStruct((SEG_PAD,), jnp.float32),
        mesh=_mesh,
        compiler_params=pltpu.CompilerParams(needs_layout_passes=False),
        scratch_types=[
            pltpu.VMEM((NW * COLS,), jnp.float32),
            pltpu.VMEM((COLS,), jnp.float32),
        ],
    )
    def _red(part_h, out_h, buf, ob):
        wid = _wid()
        c0 = wid * COLS

        def row(r, _):
            pltpu.sync_copy(part_h.at[pl.ds(r * SEG_PAD + c0, COLS)],
                            buf.at[pl.ds(r * COLS, COLS)])
            return _
        lax.fori_loop(0, NW, row, None)

        def col(g, _):
            acc = buf[pl.ds(g * 16, 16)]
            for r in range(1, NW):
                v = buf[pl.ds(r * COLS + g * 16, 16)]
                acc = jnp.maximum(acc, v) if is_max else acc + v
            ob[pl.ds(g * 16, 16)] = acc
            return _
        lax.fori_loop(0, COLS // 16, col, None)
        pltpu.sync_copy(ob, out_h.at[pl.ds(c0, COLS)])

    return _red


_k2 = _make_reduce(True)
_k4 = _make_reduce(False)


S3 = 2048
NSC3 = EPW // S3


@functools.partial(
    pl.kernel,
    out_type=[
        jax.ShapeDtypeStruct((E_PAD,), jnp.float32),
        jax.ShapeDtypeStruct((NW * SEG_PAD,), jnp.float32),
    ],
    mesh=_mesh,
    compiler_params=pltpu.CompilerParams(needs_layout_passes=False),
    scratch_types=[
        pltpu.VMEM((S3,), jnp.int32),
        pltpu.VMEM((S3,), jnp.float32),
        pltpu.VMEM((S3,), jnp.float32),
        pltpu.VMEM((SEG_PAD,), jnp.float32),
        pltpu.VMEM((SEG_PAD,), jnp.float32),
    ],
)
def _k3(dst_h, norm_h, max_h, ex_h, psum_h, dstb, nb, exb, mtab, psum):
    wid = _wid()
    pltpu.sync_copy(max_h, mtab)

    def zero(i, _):
        psum[pl.ds(i * 16, 16)] = jnp.zeros((16,), jnp.float32)
        return _
    lax.fori_loop(0, SEG_PAD // 16, zero, None)

    def sch(si, _):
        base = wid * EPW + si * S3
        pltpu.sync_copy(dst_h.at[pl.ds(base, S3)], dstb)
        pltpu.sync_copy(norm_h.at[pl.ds(base, S3)], nb)

        def grp(g, _):
            dv = dstb[pl.ds(g * 16, 16)]
            mv = plsc.load_gather(mtab, [dv])
            nv = nb[pl.ds(g * 16, 16)]
            ev = jnp.exp(nv - mv)
            exb[pl.ds(g * 16, 16)] = ev
            _seg_accum(psum, dv, ev, False)
            return _
        lax.fori_loop(0, S3 // 16, grp, None)

        pltpu.sync_copy(exb, ex_h.at[pl.ds(base, S3)])
        return _
    lax.fori_loop(0, NSC3, sch, None)
    pltpu.sync_copy(psum, psum_h.at[pl.ds(wid * SEG_PAD, SEG_PAD)])


C5 = 64
S5 = 1024
CPS5 = S5 // C5
NSC5 = EPW // S5


@functools.partial(
    pl.kernel,
    out_type=jax.ShapeDtypeStruct((NCORE, SEG_PAD, H), jnp.float32),
    mesh=_mesh,
    compiler_params=pltpu.CompilerParams(needs_layout_passes=False),
    scratch_types=[
        pltpu.VMEM((S5,), jnp.int32),
        pltpu.VMEM((S5,), jnp.int32),
        pltpu.VMEM((S5,), jnp.float32),
        pltpu.VMEM((C5,), jnp.int32),
        pltpu.VMEM((C5,), jnp.int32),
        pltpu.VMEM((C5, H), jnp.float32),
        pltpu.VMEM((C5, H), jnp.float32),
        pltpu.VMEM((C5, H), jnp.float32),
        pltpu.VMEM((C5, H), jnp.float32),
        pltpu.VMEM((SEG_PAD,), jnp.float32),
        pltpu.VMEM_SHARED((SEG_PAD, H), jnp.float32),
        pltpu.SemaphoreType.DMA,
        pltpu.SemaphoreType.DMA,
        pltpu.SemaphoreType.DMA,
    ],
)
def _k5(src_h, dst_h, rel_h, ex_h, ssum_h, ent_h, rele_h, out_h,
        srcb, relb, exb, dstb0, dstb1, bs0, br0, bs1, br1, stab, acc,
        semA, semB, semS):
    cid = lax.axis_index("c")
    sid = lax.axis_index("s")
    wid = sid * NCORE + cid

    def zrow(i, _):
        for k in range(H // 16):
            bs0[i, pl.ds(k * 16, 16)] = jnp.zeros((16,), jnp.float32)
        return _
    lax.fori_loop(0, C5, zrow, None)
    for j in range(RPS // C5):
        pltpu.sync_copy(bs0, acc.at[pl.ds(sid * RPS + j * C5, C5), :])
    plsc.subcore_barrier()

    pltpu.sync_copy(ssum_h, stab)

    def comp_chunk(bs, br, dstb, off):
        def grp(g, _):
            dv = dstb[pl.ds(g * 16, 16)]
            sv = plsc.load_gather(stab, [dv])
            alv = exb[pl.ds(off + g * 16, 16)] / sv
            for j in range(16):
                e = g * 16 + j
                a = _perm(alv, jnp.full((16,), j, jnp.int32))
                for k in range(H // 16):
                    s = bs[e, pl.ds(k * 16, 16)]
                    r = br[e, pl.ds(k * 16, 16)]
                    bs[e, pl.ds(k * 16, 16)] = (s + r) * a
            return _
        lax.fori_loop(0, C5 // 16, grp, None)

    def sch(si, _):
        sbase = wid * EPW + si * S5
        pltpu.sync_copy(src_h.at[pl.ds(sbase, S5)], srcb)
        pltpu.sync_copy(rel_h.at[pl.ds(sbase, S5)], relb)
        pltpu.sync_copy(ex_h.at[pl.ds(sbase, S5)], exb)

        def pair(p, _):
            o0 = (2 * p) * C5
            o1 = o0 + C5
            a1 = pltpu.async_copy(ent_h.at[srcb.at[pl.ds(o0, C5)]], bs0, semA)
            a2 = pltpu.async_copy(rele_h.at[relb.at[pl.ds(o0, C5)]], br0, semA)
            a3 = pltpu.async_copy(dst_h.at[pl.ds(sbase + o0, C5)], dstb0, semA)
            b1 = pltpu.async_copy(ent_h.at[srcb.at[pl.ds(o1, C5)]], bs1, semB)
            b2 = pltpu.async_copy(rele_h.at[relb.at[pl.ds(o1, C5)]], br1, semB)
            b3 = pltpu.async_copy(dst_h.at[pl.ds(sbase + o1, C5)], dstb1, semB)
            a1.wait()
            a2.wait()
            a3.wait()
            comp_chunk(bs0, br0, dstb0, o0)
            s0 = pltpu.async_copy(bs0, acc.at[dstb0], semS, add=True)
            b1.wait()
            b2.wait()
            b3.wait()
            comp_chunk(bs1, br1, dstb1, o1)
            s1 = pltpu.async_copy(bs1, acc.at[dstb1], semS, add=True)
            s0.wait()
            s1.wait()
            return _
        lax.fori_loop(0, CPS5 // 2, pair, None)
        return _
    lax.fori_loop(0, NSC5, sch, None)

    plsc.subcore_barrier()
    pltpu.sync_copy(acc.at[pl.ds(sid * RPS, RPS), :],
                    out_h.at[cid, pl.ds(sid * RPS, RPS), :])


def _mm_body(a_ref, b_ref, w_ref, o_ref):
    x = a_ref[...] + b_ref[...]
    o_ref[...] = jnp.tanh(jnp.dot(x, w_ref[...],
                                  preferred_element_type=jnp.float32))


def _tc_mm(p0, p1, w):
    blk = 1024
    return pl.pallas_call(
        _mm_body,
        grid=(SEG_PAD // blk,),
        in_specs=[
            pl.BlockSpec((blk, H), lambda i: (i, 0)),
            pl.BlockSpec((blk, H), lambda i: (i, 0)),
            pl.BlockSpec((H, H), lambda i: (0, 0)),
        ],
        out_specs=pl.BlockSpec((blk, H), lambda i: (i, 0)),
        out_shape=jax.ShapeDtypeStruct((SEG_PAD, H), jnp.float32),
    )(p0, p1, w)


@jax.jit
def kernel(ent_emb, rel_emb, edge_index, rel_id, neigh_w):
    src = edge_index[0].astype(jnp.int32)
    dst = edge_index[1].astype(jnp.int32)
    rel = rel_id.astype(jnp.int32)
    npad = E_PAD - N_EDGE
    src_p = jnp.concatenate([src, jnp.zeros((npad,), jnp.int32)])
    dst_p = jnp.concatenate([dst, jnp.full((npad,), N_ENT, jnp.int32)])
    rel_p = jnp.concatenate([rel, jnp.zeros((npad,), jnp.int32)])
    ent_p = jnp.pad(ent_emb, ((0, SEG_PAD - N_ENT), (0, 0)))

    norm, pmaxu = _k1(src_p, dst_p, rel_p, ent_p, rel_emb)
    segmax = _k2(pmaxu)
    ex, psum = _k3(dst_p, norm, segmax)
    segsum = _k4(psum)
    parts = _k5(src_p, dst_p, rel_p, ex, segsum, ent_p, rel_emb)
    out = _tc_mm(parts[0], parts[1], neigh_w)
    return out[:N_ENT]


# speedup vs baseline: 3.6807x; 1.0137x over previous
import functools
import jax
import jax.numpy as jnp
from jax import lax
from jax.experimental import pallas as pl
from jax.experimental.pallas import tpu as pltpu
from jax.experimental.pallas import tpu_sc as plsc

N_ENT = 10000
H = 128
N_REL = 475
N_EDGE = 320000

NCORE = 2
NSUB = 16
NW = NCORE * NSUB
EPW = 10240
E_PAD = NW * EPW
SEG_PAD = 10240
C = 128
NCH = EPW // C
COLS = SEG_PAD // NW
RPS = SEG_PAD // NSUB

_mesh = plsc.VectorSubcoreMesh(core_axis_name="c", subcore_axis_name="s")


def _wid():
    return lax.axis_index("s") * NCORE + lax.axis_index("c")


def _perm(v, idx):
    return v.at[idx].get(mode="promise_in_bounds")


def _seg_accum(tab, dv, uv, is_max):
    k, v = plsc.sort_key_val(dv, uv)
    iota = lax.iota(jnp.int32, 16)
    for off in (1, 2, 4, 8):
        idx = jnp.maximum(iota - off, 0)
        kp = _perm(k, idx)
        vp = _perm(v, idx)
        same = (kp == k) & (iota >= off)
        cmb = jnp.maximum(v, vp) if is_max else v + vp
        v = jnp.where(same, cmb, v)
    knext = _perm(k, jnp.minimum(iota + 1, 15))
    last = (knext != k) | (iota == 15)
    cur = plsc.load_gather(tab, [k])
    new = jnp.maximum(cur, v) if is_max else cur + v
    plsc.store_scatter(tab, [k], new, mask=last)


S1 = 2048
CPS1 = S1 // C
NSC1 = EPW // S1


@functools.partial(
    pl.kernel,
    out_type=[
        jax.ShapeDtypeStruct((E_PAD,), jnp.float32),
        jax.ShapeDtypeStruct((NW * SEG_PAD,), jnp.float32),
    ],
    mesh=_mesh,
    compiler_params=pltpu.CompilerParams(needs_layout_passes=False),
    scratch_types=[
        pltpu.VMEM((S1,), jnp.int32),
        pltpu.VMEM((S1,), jnp.int32),
        pltpu.VMEM((S1,), jnp.int32),
        pltpu.VMEM((C, H), jnp.float32),
        pltpu.VMEM((C, H), jnp.float32),
        pltpu.VMEM((C, H), jnp.float32),
        pltpu.VMEM((C, H), jnp.float32),
        pltpu.VMEM((C, H), jnp.float32),
        pltpu.VMEM((C, H), jnp.float32),
        pltpu.VMEM((S1,), jnp.float32),
        pltpu.VMEM((SEG_PAD,), jnp.float32),
        pltpu.SemaphoreType.DMA,
        pltpu.SemaphoreType.DMA,
    ],
)
def _k1(src_h, dst_h, rel_h, ent_h, rele_h, norm_h, pmax_h,
        srcb, dstb, relb, bs0, br0, bd0, bs1, br1, bd1, nb, pmaxu,
        semA, semB):
    wid = _wid()

    def zero(i, _):
        pmaxu[pl.ds(i * 16, 16)] = jnp.full((16,), -3.0e38, jnp.float32)
        return _
    lax.fori_loop(0, SEG_PAD // 16, zero, None)

    def dot_chunk(bs, br, bd, off):
        def grp(g, _):
            iota = lax.iota(jnp.int32, 16)
            nvec = jnp.zeros((16,), jnp.float32)
            for j in range(16):
                e = g * 16 + j
                acc = jnp.zeros((16,), jnp.float32)
                for k in range(H // 16):
                    s = bs[e, pl.ds(k * 16, 16)]
                    r = br[e, pl.ds(k * 16, 16)]
                    d = bd[e, pl.ds(k * 16, 16)]
                    acc = acc + (s + r) * d
                for o in (8, 4, 2, 1):
                    acc = acc + _perm(acc, (iota + o) & 15)
                nvec = jnp.where(iota == j, acc, nvec)
            nb[pl.ds(off + g * 16, 16)] = nvec
            _seg_accum(pmaxu, dstb[pl.ds(off + g * 16, 16)], nvec, True)
            return _
        lax.fori_loop(0, C // 16, grp, None)

    def sch(si, _):
        sbase = wid * EPW + si * S1
        pltpu.sync_copy(src_h.at[pl.ds(sbase, S1)], srcb)
        pltpu.sync_copy(dst_h.at[pl.ds(sbase, S1)], dstb)
        pltpu.sync_copy(rel_h.at[pl.ds(sbase, S1)], relb)

        def pair(p, _):
            o0 = (2 * p) * C
            o1 = o0 + C
            a1 = pltpu.async_copy(ent_h.at[srcb.at[pl.ds(o0, C)]], bs0, semA)
            a2 = pltpu.async_copy(rele_h.at[relb.at[pl.ds(o0, C)]], br0, semA)
            a3 = pltpu.async_copy(ent_h.at[dstb.at[pl.ds(o0, C)]], bd0, semA)
            b1 = pltpu.async_copy(ent_h.at[srcb.at[pl.ds(o1, C)]], bs1, semB)
            b2 = pltpu.async_copy(rele_h.at[relb.at[pl.ds(o1, C)]], br1, semB)
            b3 = pltpu.async_copy(ent_h.at[dstb.at[pl.ds(o1, C)]], bd1, semB)
            a1.wait()
            a2.wait()
            a3.wait()
            dot_chunk(bs0, br0, bd0, o0)
            b1.wait()
            b2.wait()
            b3.wait()
            dot_chunk(bs1, br1, bd1, o1)
            return _
        lax.fori_loop(0, CPS1 // 2, pair, None)
        pltpu.sync_copy(nb, norm_h.at[pl.ds(sbase, S1)])
        return _
    lax.fori_loop(0, NSC1, sch, None)
    pltpu.sync_copy(pmaxu, pmax_h.at[pl.ds(wid * SEG_PAD, SEG_PAD)])


def _make_reduce(is_max):
    @functools.partial(
        pl.kernel,
        out_type=jax.ShapeDtypeStruct((SEG_PAD,), jnp.float32),
        mesh=_mesh,
        compiler_params=pltpu.CompilerParams(needs_layout_passes=False),
        scratch_types=[
            pltpu.VMEM((NW * COLS,), jnp.float32),
            pltpu.VMEM((COLS,), jnp.float32),
            pltpu.SemaphoreType.DMA,
        ],
    )
    def _red(part_h, out_h, buf, ob, sem):
        wid = _wid()
        c0 = wid * COLS

        cps = [pltpu.async_copy(part_h.at[pl.ds(r * SEG_PAD + c0, COLS)],
                                buf.at[pl.ds(r * COLS, COLS)], sem)
               for r in range(NW)]
        for cp in cps:
            cp.wait()

        def col(g, _):
            acc = buf[pl.ds(g * 16, 16)]
            for r in range(1, NW):
                v = buf[pl.ds(r * COLS + g * 16, 16)]
                acc = jnp.maximum(acc, v) if is_max else acc + v
            ob[pl.ds(g * 16, 16)] = acc
            return _
        lax.fori_loop(0, COLS // 16, col, None)
        pltpu.sync_copy(ob, out_h.at[pl.ds(c0, COLS)])

    return _red


_k2 = _make_reduce(True)
_k4 = _make_reduce(False)


S3 = 2048
NSC3 = EPW // S3


@functools.partial(
    pl.kernel,
    out_type=[
        jax.ShapeDtypeStruct((E_PAD,), jnp.float32),
        jax.ShapeDtypeStruct((NW * SEG_PAD,), jnp.float32),
    ],
    mesh=_mesh,
    compiler_params=pltpu.CompilerParams(needs_layout_passes=False),
    scratch_types=[
        pltpu.VMEM((S3,), jnp.int32),
        pltpu.VMEM((S3,), jnp.float32),
        pltpu.VMEM((S3,), jnp.float32),
        pltpu.VMEM((SEG_PAD,), jnp.float32),
        pltpu.VMEM((SEG_PAD,), jnp.float32),
    ],
)
def _k3(dst_h, norm_h, max_h, ex_h, psum_h, dstb, nb, exb, mtab, psum):
    wid = _wid()
    pltpu.sync_copy(max_h, mtab)

    def zero(i, _):
        psum[pl.ds(i * 16, 16)] = jnp.zeros((16,), jnp.float32)
        return _
    lax.fori_loop(0, SEG_PAD // 16, zero, None)

    def sch(si, _):
        base = wid * EPW + si * S3
        pltpu.sync_copy(dst_h.at[pl.ds(base, S3)], dstb)
        pltpu.sync_copy(norm_h.at[pl.ds(base, S3)], nb)

        def grp(g, _):
            dv = dstb[pl.ds(g * 16, 16)]
            mv = plsc.load_gather(mtab, [dv])
            nv = nb[pl.ds(g * 16, 16)]
            ev = jnp.exp(nv - mv)
            exb[pl.ds(g * 16, 16)] = ev
            _seg_accum(psum, dv, ev, False)
            return _
        lax.fori_loop(0, S3 // 16, grp, None)

        pltpu.sync_copy(exb, ex_h.at[pl.ds(base, S3)])
        return _
    lax.fori_loop(0, NSC3, sch, None)
    pltpu.sync_copy(psum, psum_h.at[pl.ds(wid * SEG_PAD, SEG_PAD)])


C5 = 64
S5 = 1024
CPS5 = S5 // C5
NSC5 = EPW // S5


@functools.partial(
    pl.kernel,
    out_type=jax.ShapeDtypeStruct((NCORE, SEG_PAD, H), jnp.float32),
    mesh=_mesh,
    compiler_params=pltpu.CompilerParams(needs_layout_passes=False),
    scratch_types=[
        pltpu.VMEM((S5,), jnp.int32),
        pltpu.VMEM((S5,), jnp.int32),
        pltpu.VMEM((S5,), jnp.float32),
        pltpu.VMEM((C5,), jnp.int32),
        pltpu.VMEM((C5,), jnp.int32),
        pltpu.VMEM((C5, H), jnp.float32),
        pltpu.VMEM((C5, H), jnp.float32),
        pltpu.VMEM((C5, H), jnp.float32),
        pltpu.VMEM((C5, H), jnp.float32),
        pltpu.VMEM((SEG_PAD,), jnp.float32),
        pltpu.VMEM_SHARED((SEG_PAD, H), jnp.float32),
        pltpu.SemaphoreType.DMA,
        pltpu.SemaphoreType.DMA,
        pltpu.SemaphoreType.DMA,
    ],
)
def _k5(src_h, dst_h, rel_h, ex_h, ssum_h, ent_h, rele_h, out_h,
        srcb, relb, exb, dstb0, dstb1, bs0, br0, bs1, br1, stab, acc,
        semA, semB, semS):
    cid = lax.axis_index("c")
    sid = lax.axis_index("s")
    wid = sid * NCORE + cid

    def zrow(i, _):
        for k in range(H // 16):
            bs0[i, pl.ds(k * 16, 16)] = jnp.zeros((16,), jnp.float32)
        return _
    lax.fori_loop(0, C5, zrow, None)
    for j in range(RPS // C5):
        pltpu.sync_copy(bs0, acc.at[pl.ds(sid * RPS + j * C5, C5), :])
    plsc.subcore_barrier()

    pltpu.sync_copy(ssum_h, stab)

    def comp_chunk(bs, br, dstb, off):
        def grp(g, _):
            dv = dstb[pl.ds(g * 16, 16)]
            sv = plsc.load_gather(stab, [dv])
            alv = exb[pl.ds(off + g * 16, 16)] / sv
            for j in range(16):
                e = g * 16 + j
                a = _perm(alv, jnp.full((16,), j, jnp.int32))
                for k in range(H // 16):
                    s = bs[e, pl.ds(k * 16, 16)]
                    r = br[e, pl.ds(k * 16, 16)]
                    bs[e, pl.ds(k * 16, 16)] = (s + r) * a
            return _
        lax.fori_loop(0, C5 // 16, grp, None)

    def sch(si, _):
        sbase = wid * EPW + si * S5
        pltpu.sync_copy(src_h.at[pl.ds(sbase, S5)], srcb)
        pltpu.sync_copy(rel_h.at[pl.ds(sbase, S5)], relb)
        pltpu.sync_copy(ex_h.at[pl.ds(sbase, S5)], exb)

        def pair(p, _):
            o0 = (2 * p) * C5
            o1 = o0 + C5
            a1 = pltpu.async_copy(ent_h.at[srcb.at[pl.ds(o0, C5)]], bs0, semA)
            a2 = pltpu.async_copy(rele_h.at[relb.at[pl.ds(o0, C5)]], br0, semA)
            a3 = pltpu.async_copy(dst_h.at[pl.ds(sbase + o0, C5)], dstb0, semA)
            b1 = pltpu.async_copy(ent_h.at[srcb.at[pl.ds(o1, C5)]], bs1, semB)
            b2 = pltpu.async_copy(rele_h.at[relb.at[pl.ds(o1, C5)]], br1, semB)
            b3 = pltpu.async_copy(dst_h.at[pl.ds(sbase + o1, C5)], dstb1, semB)
            a1.wait()
            a2.wait()
            a3.wait()
            comp_chunk(bs0, br0, dstb0, o0)
            s0 = pltpu.async_copy(bs0, acc.at[dstb0], semS, add=True)
            b1.wait()
            b2.wait()
            b3.wait()
            comp_chunk(bs1, br1, dstb1, o1)
            s1 = pltpu.async_copy(bs1, acc.at[dstb1], semS, add=True)
            s0.wait()
            s1.wait()
            return _
        lax.fori_loop(0, CPS5 // 2, pair, None)
        return _
    lax.fori_loop(0, NSC5, sch, None)

    plsc.subcore_barrier()
    pltpu.sync_copy(acc.at[pl.ds(sid * RPS, RPS), :],
                    out_h.at[cid, pl.ds(sid * RPS, RPS), :])


def _mm_body(a_ref, b_ref, w_ref, o_ref):
    x = a_ref[...] + b_ref[...]
    o_ref[...] = jnp.tanh(jnp.dot(x, w_ref[...],
                                  preferred_element_type=jnp.float32))


def _tc_mm(p0, p1, w):
    blk = 1024
    return pl.pallas_call(
        _mm_body,
        grid=(SEG_PAD // blk,),
        in_specs=[
            pl.BlockSpec((blk, H), lambda i: (i, 0)),
            pl.BlockSpec((blk, H), lambda i: (i, 0)),
            pl.BlockSpec((H, H), lambda i: (0, 0)),
        ],
        out_specs=pl.BlockSpec((blk, H), lambda i: (i, 0)),
        out_shape=jax.ShapeDtypeStruct((SEG_PAD, H), jnp.float32),
    )(p0, p1, w)


@jax.jit
def kernel(ent_emb, rel_emb, edge_index, rel_id, neigh_w):
    src = edge_index[0].astype(jnp.int32)
    dst = edge_index[1].astype(jnp.int32)
    rel = rel_id.astype(jnp.int32)
    npad = E_PAD - N_EDGE
    src_p = jnp.concatenate([src, jnp.zeros((npad,), jnp.int32)])
    dst_p = jnp.concatenate([dst, jnp.full((npad,), N_ENT, jnp.int32)])
    rel_p = jnp.concatenate([rel, jnp.zeros((npad,), jnp.int32)])
    ent_p = jnp.pad(ent_emb, ((0, SEG_PAD - N_ENT), (0, 0)))

    norm, pmaxu = _k1(src_p, dst_p, rel_p, ent_p, rel_emb)
    segmax = _k2(pmaxu)
    ex, psum = _k3(dst_p, norm, segmax)
    segsum = _k4(psum)
    parts = _k5(src_p, dst_p, rel_p, ex, segsum, ent_p, rel_emb)
    out = _tc_mm(parts[0], parts[1], neigh_w)
    return out[:N_ENT]
